# baseline (speedup 1.0000x reference)
import functools

import jax
import jax.numpy as jnp
from jax import lax
from jax.experimental import pallas as pl
from jax.experimental.pallas import tpu as pltpu
from jax.experimental.pallas import tpu_sc as plsc

B, S, E, H, A, C = 1024, 200, 64, 64, 64, 16
N = B * S
NW = 32
ROWS_PER_W = N // NW
CHUNK = 128
NCHUNK = ROWS_PER_W // CHUNK
BB = 128
ZROWS = 64


def _sc_gather(idx3d, table):
    mesh = plsc.VectorSubcoreMesh(core_axis_name="c", subcore_axis_name="s")

    @functools.partial(
        pl.kernel,
        mesh=mesh,
        out_type=jax.ShapeDtypeStruct((NW * NCHUNK, CHUNK, 128), jnp.float32),
        scratch_types=[
            pltpu.VMEM((NCHUNK, CHUNK), jnp.int32),
            pltpu.VMEM((CHUNK, 128), jnp.float32),
            pltpu.SemaphoreType.DMA,
        ],
    )
    def k(idx_hbm, table_hbm, out_hbm, idx_v, rows_v, sem):
        wid = lax.axis_index("s") * 2 + lax.axis_index("c")
        pltpu.sync_copy(idx_hbm.at[wid], idx_v)

        def body(j, carry):
            pltpu.async_copy(table_hbm.at[idx_v.at[j]], rows_v, sem).wait()
            pltpu.sync_copy(rows_v, out_hbm.at[wid * NCHUNK + j])
            return carry

        lax.fori_loop(0, NCHUNK, body, 0)

    return k(idx3d, table)


def _attn_scores(x_flat, attW, attb2, attV2, attvb2):
    def body(x_ref, w_ref, b_ref, v_ref, vb_ref, z_ref):
        x = x_ref[...]
        u = jnp.tanh(
            jnp.dot(x, w_ref[...], preferred_element_type=jnp.float32)
            + b_ref[...])
        z = jnp.sum(u * v_ref[...], axis=-1) + vb_ref[0, 0]
        z_ref[...] = z.reshape(ZROWS, 128)

    rows = ZROWS * 128
    grid = N // rows
    return pl.pallas_call(
        body,
        grid=(grid,),
        in_specs=[
            pl.BlockSpec((rows, 128), lambda i: (i, 0)),
            pl.BlockSpec((128, A), lambda i: (0, 0)),
            pl.BlockSpec((1, A), lambda i: (0, 0)),
            pl.BlockSpec((1, A), lambda i: (0, 0)),
            pl.BlockSpec((1, 1), lambda i: (0, 0)),
        ],
        out_specs=pl.BlockSpec((ZROWS, 128), lambda i: (i, 0)),
        out_shape=jax.ShapeDtypeStruct((N // 128, 128), jnp.float32),
    )(x_flat, attW, attb2, attV2, attvb2)


def _bilstm_head(x_tm, ids_bm, attn, W_f, U_f, bf2, W_b, U_b, bb2,
                 dWf, dWb, db2):
    def body(x_ref, ids_ref, attn_ref, wf_ref, uf_ref, bf_ref,
             wb_ref, ub_ref, bb_ref, dwf_ref, dwb_ref, db_ref, out_ref):
        attn_blk = attn_ref[...]
        ids_blk = ids_ref[...]
        mx = jnp.max(attn_blk, axis=-1, keepdims=True)
        ex = jnp.exp(attn_blk - mx)
        alpha = ex / jnp.sum(ex, axis=-1, keepdims=True)
        wf = wf_ref[...]
        uf = uf_ref[...]
        bf = bf_ref[...]
        wb = wb_ref[...]
        ub = ub_ref[...]
        bb = bb_ref[...]
        lane_iota = lax.broadcasted_iota(jnp.int32, (1, S), 1)

        def gates(xt, h, c, W, U, bias):
            zg = (jnp.dot(xt, W, preferred_element_type=jnp.float32)
                  + jnp.dot(h, U, preferred_element_type=jnp.float32) + bias)
            ig = jax.nn.sigmoid(zg[:, 0:H])
            fg = jax.nn.sigmoid(zg[:, H:2 * H])
            gg = jnp.tanh(zg[:, 2 * H:3 * H])
            og = jax.nn.sigmoid(zg[:, 3 * H:4 * H])
            c_new = fg * c + ig * gg
            h_new = og * jnp.tanh(c_new)
            return h_new, c_new

        def step(t, carry):
            hf, cf, of, hb, cb, ob, acc = carry
            xt = x_ref[t]
            sel_t = (lane_iota == t)
            mt = jnp.sum(jnp.where(sel_t, ids_blk, 0),
                         axis=-1, keepdims=True) != 0
            hfn, cfn = gates(xt, hf, cf, wf, uf, bf)
            hf = jnp.where(mt, hfn, hf)
            cf = jnp.where(mt, cfn, cf)
            of = jnp.where(mt, hfn, of)
            a_t = jnp.sum(jnp.where(sel_t, alpha, 0.0),
                          axis=-1, keepdims=True)
            acc = acc + jnp.dot(of * a_t, dwf_ref[t],
                                preferred_element_type=jnp.float32)
            tr = S - 1 - t
            xr = x_ref[tr]
            sel_r = (lane_iota == tr)
            mr = jnp.sum(jnp.where(sel_r, ids_blk, 0),
                         axis=-1, keepdims=True) != 0
            hbn, cbn = gates(xr, hb, cb, wb, ub, bb)
            hb = jnp.where(mr, hbn, hb)
            cb = jnp.where(mr, cbn, cb)
            ob = jnp.where(mr, hbn, ob)
            a_r = jnp.sum(jnp.where(sel_r, alpha, 0.0),
                          axis=-1, keepdims=True)
            acc = acc + jnp.dot(ob * a_r, dwb_ref[tr],
                                preferred_element_type=jnp.float32)
            return (hf, cf, of, hb, cb, ob, acc)

        z = jnp.zeros((BB, H), jnp.float32)
        acc0 = jnp.zeros((BB, C), jnp.float32)
        carry = lax.fori_loop(0, S, step, (z, z, z, z, z, z, acc0))
        logits = carry[6] + db_ref[...]
        lm = jnp.max(logits, axis=-1, keepdims=True)
        el = jnp.exp(logits - lm)
        out_ref[...] = el / jnp.sum(el, axis=-1, keepdims=True)

    nb = B // BB
    return pl.pallas_call(
        body,
        grid=(nb,),
        in_specs=[
            pl.BlockSpec((S, BB, 128), lambda i: (0, i, 0)),
            pl.BlockSpec((BB, S), lambda i: (i, 0)),
            pl.BlockSpec((BB, S), lambda i: (i, 0)),
            pl.BlockSpec((128, 4 * H), lambda i: (0, 0)),
            pl.BlockSpec((H, 4 * H), lambda i: (0, 0)),
            pl.BlockSpec((1, 4 * H), lambda i: (0, 0)),
            pl.BlockSpec((128, 4 * H), lambda i: (0, 0)),
            pl.BlockSpec((H, 4 * H), lambda i: (0, 0)),
            pl.BlockSpec((1, 4 * H), lambda i: (0, 0)),
            pl.BlockSpec((S, H, C), lambda i: (0, 0, 0)),
            pl.BlockSpec((S, H, C), lambda i: (0, 0, 0)),
            pl.BlockSpec((1, C), lambda i: (0, 0)),
        ],
        out_specs=pl.BlockSpec((BB, C), lambda i: (i, 0)),
        out_shape=jax.ShapeDtypeStruct((B, C), jnp.float32),
    )(x_tm, ids_bm, attn, W_f, U_f, bf2, W_b, U_b, bb2, dWf, dWb, db2)


def kernel(inputs, emb, W_f, U_f, b_f, W_b, U_b, b_b,
           attW, attb, attV, attvb, dW, db):
    ids_tm = inputs.T
    idx3d = ids_tm.reshape(NW, NCHUNK, CHUNK)
    embp = jnp.pad(emb, ((0, 0), (0, 128 - E)))
    x2 = _sc_gather(idx3d, embp)
    x_flat = x2.reshape(N, 128)
    attWp = jnp.pad(attW, ((0, 128 - E), (0, 0)))
    z2d = _attn_scores(x_flat, attWp, attb.reshape(1, A),
                       attV.reshape(1, A), attvb.reshape(1, 1))
    z_tm = z2d.reshape(S, B)
    attn = z_tm.T.reshape(-1).reshape(S, B).T
    x_tm = x2.reshape(S, B, 128)
    dW3 = dW.reshape(S, 2 * H, C)
    W_fp = jnp.pad(W_f, ((0, 128 - E), (0, 0)))
    W_bp = jnp.pad(W_b, ((0, 128 - E), (0, 0)))
    out = _bilstm_head(x_tm, inputs, attn,
                       W_fp, U_f, b_f.reshape(1, -1),
                       W_bp, U_b, b_b.reshape(1, -1),
                       dW3[:, :H, :], dW3[:, H:, :], db.reshape(1, -1))
    return out


# speedup vs baseline: 1.5238x; 1.5238x over previous
import functools

import jax
import jax.numpy as jnp
from jax import lax
from jax.experimental import pallas as pl
from jax.experimental.pallas import tpu as pltpu
from jax.experimental.pallas import tpu_sc as plsc

B, S, E, H, A, C = 1024, 200, 64, 64, 64, 16
N = B * S
NW = 32
ROWS_PER_W = N // NW
CHUNK = 128
NCHUNK = ROWS_PER_W // CHUNK
BB = 128
ZROWS = 64


def _sc_gather(idx3d, table):
    mesh = plsc.VectorSubcoreMesh(core_axis_name="c", subcore_axis_name="s")

    @functools.partial(
        pl.kernel,
        mesh=mesh,
        out_type=jax.ShapeDtypeStruct((NW * NCHUNK, CHUNK, 128), jnp.float32),
        scratch_types=[
            pltpu.VMEM((NCHUNK, CHUNK), jnp.int32),
            pltpu.VMEM((CHUNK, 128), jnp.float32),
            pltpu.SemaphoreType.DMA,
        ],
    )
    def k(idx_hbm, table_hbm, out_hbm, idx_v, rows_v, sem):
        wid = lax.axis_index("s") * 2 + lax.axis_index("c")
        pltpu.sync_copy(idx_hbm.at[wid], idx_v)

        def body(j, carry):
            pltpu.async_copy(table_hbm.at[idx_v.at[j]], rows_v, sem).wait()
            pltpu.sync_copy(rows_v, out_hbm.at[wid * NCHUNK + j])
            return carry

        lax.fori_loop(0, NCHUNK, body, 0)

    return k(idx3d, table)


def _attn_scores(x_flat, attW, attb2, attV2, attvb2):
    def body(x_ref, w_ref, b_ref, v_ref, vb_ref, z_ref):
        x = x_ref[...]
        u = jnp.tanh(
            jnp.dot(x, w_ref[...], preferred_element_type=jnp.float32)
            + b_ref[...])
        z = jnp.sum(u * v_ref[...], axis=-1) + vb_ref[0, 0]
        z_ref[...] = z.reshape(ZROWS, 128)

    rows = ZROWS * 128
    grid = N // rows
    return pl.pallas_call(
        body,
        grid=(grid,),
        in_specs=[
            pl.BlockSpec((rows, 128), lambda i: (i, 0)),
            pl.BlockSpec((128, A), lambda i: (0, 0)),
            pl.BlockSpec((1, A), lambda i: (0, 0)),
            pl.BlockSpec((1, A), lambda i: (0, 0)),
            pl.BlockSpec((1, 1), lambda i: (0, 0)),
        ],
        out_specs=pl.BlockSpec((ZROWS, 128), lambda i: (i, 0)),
        out_shape=jax.ShapeDtypeStruct((N // 128, 128), jnp.float32),
    )(x_flat, attW, attb2, attV2, attvb2)


def _attn_softmax_mask(attn, ids):
    def body(attn_ref, ids_ref, alpha_ref, msk_ref):
        a = attn_ref[...]
        mx = jnp.max(a, axis=-1, keepdims=True)
        ex = jnp.exp(a - mx)
        alpha_ref[...] = ex / jnp.sum(ex, axis=-1, keepdims=True)
        msk_ref[...] = (ids_ref[...] != 0).astype(jnp.float32)

    return pl.pallas_call(
        body,
        out_shape=(jax.ShapeDtypeStruct((B, S), jnp.float32),
                   jax.ShapeDtypeStruct((B, S), jnp.float32)),
    )(attn, ids)


TCH = 8
NT = S // TCH


def _bilstm_head(x_tm, alpha4, msk4, W_f, U_f, bf2, W_b, U_b, bb2,
                 dWf, dWb, db2):
    def body(xf_ref, xb_ref, af_ref, ab_ref, mf_ref, mb_ref,
             wf_ref, uf_ref, bf_ref, wb_ref, ub_ref, bb_ref,
             dwf_ref, dwb_ref, db_ref, out_ref,
             hf_s, cf_s, of_s, hb_s, cb_s, ob_s, acc_s):
        c = pl.program_id(0)

        @pl.when(c == 0)
        def _init():
            zz = jnp.zeros((B, H), jnp.float32)
            hf_s[...] = zz
            cf_s[...] = zz
            of_s[...] = zz
            hb_s[...] = zz
            cb_s[...] = zz
            ob_s[...] = zz
            acc_s[...] = jnp.zeros((B, C), jnp.float32)

        wf = wf_ref[...]
        uf = uf_ref[...]
        bf = bf_ref[...]
        wb = wb_ref[...]
        ub = ub_ref[...]
        bb = bb_ref[...]
        af = af_ref[0]
        ab = ab_ref[0]
        mf = mf_ref[0]
        mb = mb_ref[0]

        hf, cf, of = hf_s[...], cf_s[...], of_s[...]
        hb, cb, ob = hb_s[...], cb_s[...], ob_s[...]
        acc = acc_s[...]

        def gates(xt, h, cc, W, U, bias):
            zg = (jnp.dot(xt, W, preferred_element_type=jnp.float32)
                  + jnp.dot(h, U, preferred_element_type=jnp.float32) + bias)
            ig = jax.nn.sigmoid(zg[:, 0:H])
            fg = jax.nn.sigmoid(zg[:, H:2 * H])
            gg = jnp.tanh(zg[:, 2 * H:3 * H])
            og = jax.nn.sigmoid(zg[:, 3 * H:4 * H])
            c_new = fg * cc + ig * gg
            h_new = og * jnp.tanh(c_new)
            return h_new, c_new

        for tl in range(TCH):
            xt = xf_ref[tl]
            hfn, cfn = gates(xt, hf, cf, wf, uf, bf)
            m = mf[:, tl:tl + 1]
            hf = m * hfn + (1.0 - m) * hf
            cf = m * cfn + (1.0 - m) * cf
            of = m * hfn + (1.0 - m) * of
            a = af[:, tl:tl + 1]
            acc = acc + jnp.dot(of * a, dwf_ref[c * TCH + tl],
                                preferred_element_type=jnp.float32)
            tr = TCH - 1 - tl
            xr = xb_ref[tr]
            hbn, cbn = gates(xr, hb, cb, wb, ub, bb)
            mr = mb[:, tr:tr + 1]
            hb = mr * hbn + (1.0 - mr) * hb
            cb = mr * cbn + (1.0 - mr) * cb
            ob = mr * hbn + (1.0 - mr) * ob
            ar = ab[:, tr:tr + 1]
            acc = acc + jnp.dot(ob * ar, dwb_ref[(NT - 1 - c) * TCH + tr],
                                preferred_element_type=jnp.float32)

        hf_s[...], cf_s[...], of_s[...] = hf, cf, of
        hb_s[...], cb_s[...], ob_s[...] = hb, cb, ob
        acc_s[...] = acc

        @pl.when(c == NT - 1)
        def _fin():
            logits = acc + db_ref[...]
            lm = jnp.max(logits, axis=-1, keepdims=True)
            el = jnp.exp(logits - lm)
            out_ref[...] = el / jnp.sum(el, axis=-1, keepdims=True)

    return pl.pallas_call(
        body,
        grid=(NT,),
        in_specs=[
            pl.BlockSpec((TCH, B, 128), lambda c: (c, 0, 0)),
            pl.BlockSpec((TCH, B, 128), lambda c: (NT - 1 - c, 0, 0)),
            pl.BlockSpec((1, B, TCH), lambda c: (c, 0, 0)),
            pl.BlockSpec((1, B, TCH), lambda c: (NT - 1 - c, 0, 0)),
            pl.BlockSpec((1, B, TCH), lambda c: (c, 0, 0)),
            pl.BlockSpec((1, B, TCH), lambda c: (NT - 1 - c, 0, 0)),
            pl.BlockSpec((128, 4 * H), lambda c: (0, 0)),
            pl.BlockSpec((H, 4 * H), lambda c: (0, 0)),
            pl.BlockSpec((1, 4 * H), lambda c: (0, 0)),
            pl.BlockSpec((128, 4 * H), lambda c: (0, 0)),
            pl.BlockSpec((H, 4 * H), lambda c: (0, 0)),
            pl.BlockSpec((1, 4 * H), lambda c: (0, 0)),
            pl.BlockSpec((S, H, C), lambda c: (0, 0, 0)),
            pl.BlockSpec((S, H, C), lambda c: (0, 0, 0)),
            pl.BlockSpec((1, C), lambda c: (0, 0)),
        ],
        out_specs=pl.BlockSpec((B, C), lambda c: (0, 0)),
        out_shape=jax.ShapeDtypeStruct((B, C), jnp.float32),
        scratch_shapes=[pltpu.VMEM((B, H), jnp.float32)] * 6
        + [pltpu.VMEM((B, C), jnp.float32)],
    )(x_tm, x_tm, alpha4, alpha4, msk4, msk4,
      W_f, U_f, bf2, W_b, U_b, bb2, dWf, dWb, db2)


def kernel(inputs, emb, W_f, U_f, b_f, W_b, U_b, b_b,
           attW, attb, attV, attvb, dW, db):
    ids_tm = inputs.T
    idx3d = ids_tm.reshape(NW, NCHUNK, CHUNK)
    embp = jnp.pad(emb, ((0, 0), (0, 128 - E)))
    x2 = _sc_gather(idx3d, embp)
    x_flat = x2.reshape(N, 128)
    attWp = jnp.pad(attW, ((0, 128 - E), (0, 0)))
    z2d = _attn_scores(x_flat, attWp, attb.reshape(1, A),
                       attV.reshape(1, A), attvb.reshape(1, 1))
    z_tm = z2d.reshape(S, B)
    attn = z_tm.T.reshape(-1).reshape(S, B).T
    alpha, mskf = _attn_softmax_mask(attn, inputs)
    alpha4 = alpha.reshape(B, NT, TCH).transpose(1, 0, 2)
    msk4 = mskf.reshape(B, NT, TCH).transpose(1, 0, 2)
    x_tm = x2.reshape(S, B, 128)
    dW3 = dW.reshape(S, 2 * H, C)
    W_fp = jnp.pad(W_f, ((0, 128 - E), (0, 0)))
    W_bp = jnp.pad(W_b, ((0, 128 - E), (0, 0)))
    out = _bilstm_head(x_tm, alpha4, msk4,
                       W_fp, U_f, b_f.reshape(1, -1),
                       W_bp, U_b, b_b.reshape(1, -1),
                       dW3[:, :H, :], dW3[:, H:, :], db.reshape(1, -1))
    return out


# speedup vs baseline: 1.7878x; 1.1733x over previous
import functools

import jax
import jax.numpy as jnp
from jax import lax
from jax.experimental import pallas as pl
from jax.experimental.pallas import tpu as pltpu
from jax.experimental.pallas import tpu_sc as plsc

B, S, E, H, A, C = 1024, 200, 64, 64, 64, 16
N = B * S
NW = 32
ROWS_PER_W = N // NW
CHUNK = 128
NCHUNK = ROWS_PER_W // CHUNK
BB = 128
ZROWS = 64


def _sc_gather(idx3d, table):
    mesh = plsc.VectorSubcoreMesh(core_axis_name="c", subcore_axis_name="s")

    @functools.partial(
        pl.kernel,
        mesh=mesh,
        out_type=jax.ShapeDtypeStruct((NW * NCHUNK, CHUNK, 128), jnp.float32),
        scratch_types=[
            pltpu.VMEM((NCHUNK, CHUNK), jnp.int32),
            pltpu.VMEM((CHUNK, 128), jnp.float32),
            pltpu.SemaphoreType.DMA,
        ],
    )
    def k(idx_hbm, table_hbm, out_hbm, idx_v, rows_v, sem):
        wid = lax.axis_index("s") * 2 + lax.axis_index("c")
        pltpu.sync_copy(idx_hbm.at[wid], idx_v)

        def body(j, carry):
            pltpu.async_copy(table_hbm.at[idx_v.at[j]], rows_v, sem).wait()
            pltpu.sync_copy(rows_v, out_hbm.at[wid * NCHUNK + j])
            return carry

        lax.fori_loop(0, NCHUNK, body, 0)

    return k(idx3d, table)


def _attn_scores(x_flat, attW, attb2, attV2, attvb2):
    def body(x_ref, w_ref, b_ref, v_ref, vb_ref, z_ref):
        x = x_ref[...]
        u = jnp.tanh(
            jnp.dot(x, w_ref[...], preferred_element_type=jnp.float32)
            + b_ref[...])
        z = jnp.sum(u * v_ref[...], axis=-1) + vb_ref[0, 0]
        z_ref[...] = z.reshape(ZROWS, 128)

    rows = ZROWS * 128
    grid = N // rows
    return pl.pallas_call(
        body,
        grid=(grid,),
        in_specs=[
            pl.BlockSpec((rows, 128), lambda i: (i, 0)),
            pl.BlockSpec((128, A), lambda i: (0, 0)),
            pl.BlockSpec((1, A), lambda i: (0, 0)),
            pl.BlockSpec((1, A), lambda i: (0, 0)),
            pl.BlockSpec((1, 1), lambda i: (0, 0)),
        ],
        out_specs=pl.BlockSpec((ZROWS, 128), lambda i: (i, 0)),
        out_shape=jax.ShapeDtypeStruct((N // 128, 128), jnp.float32),
    )(x_flat, attW, attb2, attV2, attvb2)


def _attn_softmax_mask(attn, ids):
    def body(attn_ref, ids_ref, alpha_ref, msk_ref):
        a = attn_ref[...]
        mx = jnp.max(a, axis=-1, keepdims=True)
        ex = jnp.exp(a - mx)
        alpha_ref[...] = ex / jnp.sum(ex, axis=-1, keepdims=True)
        msk_ref[...] = (ids_ref[...] != 0).astype(jnp.float32)

    return pl.pallas_call(
        body,
        out_shape=(jax.ShapeDtypeStruct((B, S), jnp.float32),
                   jax.ShapeDtypeStruct((B, S), jnp.float32)),
    )(attn, ids)


TCH = 8
NT = S // TCH


def _bilstm_head(x_tm, alpha4, msk4, W_f, U_f, bf2, W_b, U_b, bb2,
                 dWfc, dWbc, db2):
    def body(xf_ref, xb_ref, af_ref, ab_ref, mf_ref, mb_ref,
             wf_ref, uf_ref, bf_ref, wb_ref, ub_ref, bb_ref,
             dwf_ref, dwb_ref, db_ref, out_ref,
             hf_s, cf_s, of_s, hb_s, cb_s, ob_s, acc_s):
        c = pl.program_id(0)

        @pl.when(c == 0)
        def _init():
            zz = jnp.zeros((B, H), jnp.float32)
            hf_s[...] = zz
            cf_s[...] = zz
            of_s[...] = zz
            hb_s[...] = zz
            cb_s[...] = zz
            ob_s[...] = zz
            acc_s[...] = jnp.zeros((B, C), jnp.float32)

        uf = uf_ref[...]
        ub = ub_ref[...]
        af = af_ref[0]
        ab = ab_ref[0]
        mf = mf_ref[0]
        mb = mb_ref[0]

        xf = xf_ref[...].reshape(TCH * B, 128)[:, :E]
        xwf = (jnp.dot(xf, wf_ref[...], preferred_element_type=jnp.float32)
               + bf_ref[...]).reshape(TCH, B, 4 * H)
        xb = xb_ref[...].reshape(TCH * B, 128)[:, :E]
        xwb = (jnp.dot(xb, wb_ref[...], preferred_element_type=jnp.float32)
               + bb_ref[...]).reshape(TCH, B, 4 * H)

        hf, cf, of = hf_s[...], cf_s[...], of_s[...]
        hb, cb, ob = hb_s[...], cb_s[...], ob_s[...]

        def gates(zg, cc):
            ig = jax.nn.sigmoid(zg[:, 0:H])
            fg = jax.nn.sigmoid(zg[:, H:2 * H])
            gg = jnp.tanh(zg[:, 2 * H:3 * H])
            og = jax.nn.sigmoid(zg[:, 3 * H:4 * H])
            c_new = fg * cc + ig * gg
            h_new = og * jnp.tanh(c_new)
            return h_new, c_new

        parts_f = []
        parts_b = [None] * TCH
        for tl in range(TCH):
            zgf = xwf[tl] + jnp.dot(hf, uf, preferred_element_type=jnp.float32)
            hfn, cfn = gates(zgf, cf)
            m = mf[:, tl:tl + 1] > 0.0
            hf = jnp.where(m, hfn, hf)
            cf = jnp.where(m, cfn, cf)
            of = jnp.where(m, hfn, of)
            parts_f.append(of * af[:, tl:tl + 1])
            tr = TCH - 1 - tl
            zgb = xwb[tr] + jnp.dot(hb, ub, preferred_element_type=jnp.float32)
            hbn, cbn = gates(zgb, cb)
            mr = mb[:, tr:tr + 1] > 0.0
            hb = jnp.where(mr, hbn, hb)
            cb = jnp.where(mr, cbn, cb)
            ob = jnp.where(mr, hbn, ob)
            parts_b[tr] = ob * ab[:, tr:tr + 1]

        ofcat = jnp.concatenate(parts_f, axis=1)
        obcat = jnp.concatenate(parts_b, axis=1)
        acc = (acc_s[...]
               + jnp.dot(ofcat, dwf_ref[0], preferred_element_type=jnp.float32)
               + jnp.dot(obcat, dwb_ref[0], preferred_element_type=jnp.float32))

        hf_s[...], cf_s[...], of_s[...] = hf, cf, of
        hb_s[...], cb_s[...], ob_s[...] = hb, cb, ob
        acc_s[...] = acc

        @pl.when(c == NT - 1)
        def _fin():
            logits = acc + db_ref[...]
            lm = jnp.max(logits, axis=-1, keepdims=True)
            el = jnp.exp(logits - lm)
            out_ref[...] = el / jnp.sum(el, axis=-1, keepdims=True)

    return pl.pallas_call(
        body,
        grid=(NT,),
        in_specs=[
            pl.BlockSpec((TCH, B, 128), lambda c: (c, 0, 0)),
            pl.BlockSpec((TCH, B, 128), lambda c: (NT - 1 - c, 0, 0)),
            pl.BlockSpec((1, B, TCH), lambda c: (c, 0, 0)),
            pl.BlockSpec((1, B, TCH), lambda c: (NT - 1 - c, 0, 0)),
            pl.BlockSpec((1, B, TCH), lambda c: (c, 0, 0)),
            pl.BlockSpec((1, B, TCH), lambda c: (NT - 1 - c, 0, 0)),
            pl.BlockSpec((E, 4 * H), lambda c: (0, 0)),
            pl.BlockSpec((H, 4 * H), lambda c: (0, 0)),
            pl.BlockSpec((1, 4 * H), lambda c: (0, 0)),
            pl.BlockSpec((E, 4 * H), lambda c: (0, 0)),
            pl.BlockSpec((H, 4 * H), lambda c: (0, 0)),
            pl.BlockSpec((1, 4 * H), lambda c: (0, 0)),
            pl.BlockSpec((1, TCH * H, C), lambda c: (c, 0, 0)),
            pl.BlockSpec((1, TCH * H, C), lambda c: (NT - 1 - c, 0, 0)),
            pl.BlockSpec((1, C), lambda c: (0, 0)),
        ],
        out_specs=pl.BlockSpec((B, C), lambda c: (0, 0)),
        out_shape=jax.ShapeDtypeStruct((B, C), jnp.float32),
        scratch_shapes=[pltpu.VMEM((B, H), jnp.float32)] * 6
        + [pltpu.VMEM((B, C), jnp.float32)],
    )(x_tm, x_tm, alpha4, alpha4, msk4, msk4,
      W_f, U_f, bf2, W_b, U_b, bb2, dWfc, dWbc, db2)


def kernel(inputs, emb, W_f, U_f, b_f, W_b, U_b, b_b,
           attW, attb, attV, attvb, dW, db):
    ids_tm = inputs.T
    idx3d = ids_tm.reshape(NW, NCHUNK, CHUNK)
    embp = jnp.pad(emb, ((0, 0), (0, 128 - E)))
    x2 = _sc_gather(idx3d, embp)
    x_flat = x2.reshape(N, 128)
    attWp = jnp.pad(attW, ((0, 128 - E), (0, 0)))
    z2d = _attn_scores(x_flat, attWp, attb.reshape(1, A),
                       attV.reshape(1, A), attvb.reshape(1, 1))
    z_tm = z2d.reshape(S, B)
    attn = z_tm.T.reshape(-1).reshape(S, B).T
    alpha, mskf = _attn_softmax_mask(attn, inputs)
    alpha4 = alpha.reshape(B, NT, TCH).transpose(1, 0, 2)
    msk4 = mskf.reshape(B, NT, TCH).transpose(1, 0, 2)
    x_tm = x2.reshape(S, B, 128)
    dW3 = dW.reshape(S, 2 * H, C)
    dWfc = dW3[:, :H, :].reshape(NT, TCH * H, C)
    dWbc = dW3[:, H:, :].reshape(NT, TCH * H, C)
    out = _bilstm_head(x_tm, alpha4, msk4,
                       W_f, U_f, b_f.reshape(1, -1),
                       W_b, U_b, b_b.reshape(1, -1),
                       dWfc, dWbc, db.reshape(1, -1))
    return out


# speedup vs baseline: 1.8170x; 1.0163x over previous
import functools

import jax
import jax.numpy as jnp
from jax import lax
from jax.experimental import pallas as pl
from jax.experimental.pallas import tpu as pltpu
from jax.experimental.pallas import tpu_sc as plsc

B, S, E, H, A, C = 1024, 200, 64, 64, 64, 16
N = B * S
NW = 32
ROWS_PER_W = N // NW
CHUNK = 128
NCHUNK = ROWS_PER_W // CHUNK
BB = 128
ZROWS = 64


def _sc_gather(idx3d, table):
    mesh = plsc.VectorSubcoreMesh(core_axis_name="c", subcore_axis_name="s")

    @functools.partial(
        pl.kernel,
        mesh=mesh,
        out_type=jax.ShapeDtypeStruct((NW * NCHUNK, CHUNK, 128), jnp.float32),
        scratch_types=[
            pltpu.VMEM((NCHUNK, CHUNK), jnp.int32),
            pltpu.VMEM((CHUNK, 128), jnp.float32),
            pltpu.VMEM((CHUNK, 128), jnp.float32),
            pltpu.SemaphoreType.DMA,
            pltpu.SemaphoreType.DMA,
            pltpu.SemaphoreType.DMA,
            pltpu.SemaphoreType.DMA,
        ],
    )
    def k(idx_hbm, table_hbm, out_hbm, idx_v, rows_a, rows_b, gs_a, gs_b,
          os_a, os_b):
        wid = lax.axis_index("s") * 2 + lax.axis_index("c")
        base = wid * NCHUNK
        pltpu.sync_copy(idx_hbm.at[wid], idx_v)

        pltpu.async_copy(table_hbm.at[idx_v.at[0]], rows_a, gs_a)

        def body(i, carry):
            j = i * 2

            def stage(jj, rows, gsem, osem, rows_nxt, gsem_nxt, osem_nxt):
                pltpu.make_async_copy(
                    table_hbm.at[idx_v.at[jj]], rows, gsem).wait()

                @pl.when(jj + 1 < NCHUNK)
                def _():
                    @pl.when(jj >= 1)
                    def _():
                        pltpu.make_async_copy(
                            rows_nxt, out_hbm.at[base], osem_nxt).wait()
                    pltpu.async_copy(
                        table_hbm.at[idx_v.at[jj + 1]], rows_nxt, gsem_nxt)

                pltpu.async_copy(rows, out_hbm.at[base + jj], osem)

            stage(j, rows_a, gs_a, os_a, rows_b, gs_b, os_b)
            stage(j + 1, rows_b, gs_b, os_b, rows_a, gs_a, os_a)
            return carry

        lax.fori_loop(0, NCHUNK // 2, body, 0)
        pltpu.make_async_copy(rows_a, out_hbm.at[base], os_a).wait()
        pltpu.make_async_copy(rows_b, out_hbm.at[base], os_b).wait()

    return k(idx3d, table)


def _attn_scores(x_flat, attW, attb2, attV2, attvb2):
    def body(x_ref, w_ref, b_ref, v_ref, vb_ref, z_ref):
        x = x_ref[...]
        u = jnp.tanh(
            jnp.dot(x, w_ref[...], preferred_element_type=jnp.float32)
            + b_ref[...])
        z = jnp.sum(u * v_ref[...], axis=-1) + vb_ref[0, 0]
        z_ref[...] = z.reshape(ZROWS, 128)

    rows = ZROWS * 128
    grid = N // rows
    return pl.pallas_call(
        body,
        grid=(grid,),
        in_specs=[
            pl.BlockSpec((rows, 128), lambda i: (i, 0)),
            pl.BlockSpec((128, A), lambda i: (0, 0)),
            pl.BlockSpec((1, A), lambda i: (0, 0)),
            pl.BlockSpec((1, A), lambda i: (0, 0)),
            pl.BlockSpec((1, 1), lambda i: (0, 0)),
        ],
        out_specs=pl.BlockSpec((ZROWS, 128), lambda i: (i, 0)),
        out_shape=jax.ShapeDtypeStruct((N // 128, 128), jnp.float32),
    )(x_flat, attW, attb2, attV2, attvb2)


def _attn_softmax_mask(attn, ids):
    def body(attn_ref, ids_ref, alpha_ref, msk_ref):
        a = attn_ref[...]
        mx = jnp.max(a, axis=-1, keepdims=True)
        ex = jnp.exp(a - mx)
        alpha_ref[...] = ex / jnp.sum(ex, axis=-1, keepdims=True)
        msk_ref[...] = (ids_ref[...] != 0).astype(jnp.float32)

    return pl.pallas_call(
        body,
        out_shape=(jax.ShapeDtypeStruct((B, S), jnp.float32),
                   jax.ShapeDtypeStruct((B, S), jnp.float32)),
    )(attn, ids)


TCH = 8
NT = S // TCH


def _bilstm_head(x_tm, alpha4, msk4, W_f, U_f, bf2, W_b, U_b, bb2,
                 dWfc, dWbc, db2):
    def body(xf_ref, xb_ref, af_ref, ab_ref, mf_ref, mb_ref,
             wf_ref, uf_ref, bf_ref, wb_ref, ub_ref, bb_ref,
             dwf_ref, dwb_ref, db_ref, out_ref,
             hf_s, cf_s, of_s, hb_s, cb_s, ob_s, acc_s):
        c = pl.program_id(0)

        @pl.when(c == 0)
        def _init():
            zz = jnp.zeros((B, H), jnp.float32)
            hf_s[...] = zz
            cf_s[...] = zz
            of_s[...] = zz
            hb_s[...] = zz
            cb_s[...] = zz
            ob_s[...] = zz
            acc_s[...] = jnp.zeros((B, C), jnp.float32)

        uf = uf_ref[...]
        ub = ub_ref[...]
        af = af_ref[0]
        ab = ab_ref[0]
        mf = mf_ref[0]
        mb = mb_ref[0]

        xf = xf_ref[...].reshape(TCH * B, 128)[:, :E]
        xwf = (jnp.dot(xf, wf_ref[...], preferred_element_type=jnp.float32)
               + bf_ref[...]).reshape(TCH, B, 4 * H)
        xb = xb_ref[...].reshape(TCH * B, 128)[:, :E]
        xwb = (jnp.dot(xb, wb_ref[...], preferred_element_type=jnp.float32)
               + bb_ref[...]).reshape(TCH, B, 4 * H)

        hf, cf, of = hf_s[...], cf_s[...], of_s[...]
        hb, cb, ob = hb_s[...], cb_s[...], ob_s[...]

        def gates(zg, cc):
            ig = jax.nn.sigmoid(zg[:, 0:H])
            fg = jax.nn.sigmoid(zg[:, H:2 * H])
            gg = jnp.tanh(zg[:, 2 * H:3 * H])
            og = jax.nn.sigmoid(zg[:, 3 * H:4 * H])
            c_new = fg * cc + ig * gg
            h_new = og * jnp.tanh(c_new)
            return h_new, c_new

        parts_f = []
        parts_b = [None] * TCH
        for tl in range(TCH):
            zgf = xwf[tl] + jnp.dot(hf, uf, preferred_element_type=jnp.float32)
            hfn, cfn = gates(zgf, cf)
            m = mf[:, tl:tl + 1] > 0.0
            hf = jnp.where(m, hfn, hf)
            cf = jnp.where(m, cfn, cf)
            of = jnp.where(m, hfn, of)
            parts_f.append(of * af[:, tl:tl + 1])
            tr = TCH - 1 - tl
            zgb = xwb[tr] + jnp.dot(hb, ub, preferred_element_type=jnp.float32)
            hbn, cbn = gates(zgb, cb)
            mr = mb[:, tr:tr + 1] > 0.0
            hb = jnp.where(mr, hbn, hb)
            cb = jnp.where(mr, cbn, cb)
            ob = jnp.where(mr, hbn, ob)
            parts_b[tr] = ob * ab[:, tr:tr + 1]

        ofcat = jnp.concatenate(parts_f, axis=1)
        obcat = jnp.concatenate(parts_b, axis=1)
        acc = (acc_s[...]
               + jnp.dot(ofcat, dwf_ref[0], preferred_element_type=jnp.float32)
               + jnp.dot(obcat, dwb_ref[0], preferred_element_type=jnp.float32))

        hf_s[...], cf_s[...], of_s[...] = hf, cf, of
        hb_s[...], cb_s[...], ob_s[...] = hb, cb, ob
        acc_s[...] = acc

        @pl.when(c == NT - 1)
        def _fin():
            logits = acc + db_ref[...]
            lm = jnp.max(logits, axis=-1, keepdims=True)
            el = jnp.exp(logits - lm)
            out_ref[...] = el / jnp.sum(el, axis=-1, keepdims=True)

    return pl.pallas_call(
        body,
        grid=(NT,),
        in_specs=[
            pl.BlockSpec((TCH, B, 128), lambda c: (c, 0, 0)),
            pl.BlockSpec((TCH, B, 128), lambda c: (NT - 1 - c, 0, 0)),
            pl.BlockSpec((1, B, TCH), lambda c: (c, 0, 0)),
            pl.BlockSpec((1, B, TCH), lambda c: (NT - 1 - c, 0, 0)),
            pl.BlockSpec((1, B, TCH), lambda c: (c, 0, 0)),
            pl.BlockSpec((1, B, TCH), lambda c: (NT - 1 - c, 0, 0)),
            pl.BlockSpec((E, 4 * H), lambda c: (0, 0)),
            pl.BlockSpec((H, 4 * H), lambda c: (0, 0)),
            pl.BlockSpec((1, 4 * H), lambda c: (0, 0)),
            pl.BlockSpec((E, 4 * H), lambda c: (0, 0)),
            pl.BlockSpec((H, 4 * H), lambda c: (0, 0)),
            pl.BlockSpec((1, 4 * H), lambda c: (0, 0)),
            pl.BlockSpec((1, TCH * H, C), lambda c: (c, 0, 0)),
            pl.BlockSpec((1, TCH * H, C), lambda c: (NT - 1 - c, 0, 0)),
            pl.BlockSpec((1, C), lambda c: (0, 0)),
        ],
        out_specs=pl.BlockSpec((B, C), lambda c: (0, 0)),
        out_shape=jax.ShapeDtypeStruct((B, C), jnp.float32),
        scratch_shapes=[pltpu.VMEM((B, H), jnp.float32)] * 6
        + [pltpu.VMEM((B, C), jnp.float32)],
    )(x_tm, x_tm, alpha4, alpha4, msk4, msk4,
      W_f, U_f, bf2, W_b, U_b, bb2, dWfc, dWbc, db2)


def kernel(inputs, emb, W_f, U_f, b_f, W_b, U_b, b_b,
           attW, attb, attV, attvb, dW, db):
    ids_tm = inputs.T
    idx3d = ids_tm.reshape(NW, NCHUNK, CHUNK)
    embp = jnp.pad(emb, ((0, 0), (0, 128 - E)))
    x2 = _sc_gather(idx3d, embp)
    x_flat = x2.reshape(N, 128)
    attWp = jnp.pad(attW, ((0, 128 - E), (0, 0)))
    z2d = _attn_scores(x_flat, attWp, attb.reshape(1, A),
                       attV.reshape(1, A), attvb.reshape(1, 1))
    z_tm = z2d.reshape(S, B)
    attn = z_tm.T.reshape(-1).reshape(S, B).T
    alpha, mskf = _attn_softmax_mask(attn, inputs)
    alpha4 = alpha.reshape(B, NT, TCH).transpose(1, 0, 2)
    msk4 = mskf.reshape(B, NT, TCH).transpose(1, 0, 2)
    x_tm = x2.reshape(S, B, 128)
    dW3 = dW.reshape(S, 2 * H, C)
    dWfc = dW3[:, :H, :].reshape(NT, TCH * H, C)
    dWbc = dW3[:, H:, :].reshape(NT, TCH * H, C)
    out = _bilstm_head(x_tm, alpha4, msk4,
                       W_f, U_f, b_f.reshape(1, -1),
                       W_b, U_b, b_b.reshape(1, -1),
                       dWfc, dWbc, db.reshape(1, -1))
    return out


# speedup vs baseline: 1.8273x; 1.0057x over previous
import functools

import jax
import jax.numpy as jnp
from jax import lax
from jax.experimental import pallas as pl
from jax.experimental.pallas import tpu as pltpu
from jax.experimental.pallas import tpu_sc as plsc

B, S, E, H, A, C = 1024, 200, 64, 64, 64, 16
N = B * S
NW = 32
ROWS_PER_W = N // NW
CHUNK = 128
NCHUNK = ROWS_PER_W // CHUNK
BB = 128
ZROWS = 64


def _sc_gather(idx3d, table):
    mesh = plsc.VectorSubcoreMesh(core_axis_name="c", subcore_axis_name="s")

    @functools.partial(
        pl.kernel,
        mesh=mesh,
        out_type=jax.ShapeDtypeStruct((NW * NCHUNK, CHUNK, 128), jnp.float32),
        scratch_types=[
            pltpu.VMEM((NCHUNK, CHUNK), jnp.int32),
            pltpu.VMEM((CHUNK, 128), jnp.float32),
            pltpu.VMEM((CHUNK, 128), jnp.float32),
            pltpu.SemaphoreType.DMA,
            pltpu.SemaphoreType.DMA,
            pltpu.SemaphoreType.DMA,
            pltpu.SemaphoreType.DMA,
        ],
    )
    def k(idx_hbm, table_hbm, out_hbm, idx_v, rows_a, rows_b, gs_a, gs_b,
          os_a, os_b):
        wid = lax.axis_index("s") * 2 + lax.axis_index("c")
        base = wid * NCHUNK
        pltpu.sync_copy(idx_hbm.at[wid], idx_v)

        pltpu.async_copy(table_hbm.at[idx_v.at[0]], rows_a, gs_a)

        def body(i, carry):
            j = i * 2

            def stage(jj, rows, gsem, osem, rows_nxt, gsem_nxt, osem_nxt):
                pltpu.make_async_copy(
                    table_hbm.at[idx_v.at[jj]], rows, gsem).wait()

                @pl.when(jj + 1 < NCHUNK)
                def _():
                    @pl.when(jj >= 1)
                    def _():
                        pltpu.make_async_copy(
                            rows_nxt, out_hbm.at[base], osem_nxt).wait()
                    pltpu.async_copy(
                        table_hbm.at[idx_v.at[jj + 1]], rows_nxt, gsem_nxt)

                pltpu.async_copy(rows, out_hbm.at[base + jj], osem)

            stage(j, rows_a, gs_a, os_a, rows_b, gs_b, os_b)
            stage(j + 1, rows_b, gs_b, os_b, rows_a, gs_a, os_a)
            return carry

        lax.fori_loop(0, NCHUNK // 2, body, 0)
        pltpu.make_async_copy(rows_a, out_hbm.at[base], os_a).wait()
        pltpu.make_async_copy(rows_b, out_hbm.at[base], os_b).wait()

    return k(idx3d, table)


def _attn_scores(x_flat, attW, attb2, attV2, attvb2):
    def body(x_ref, w_ref, b_ref, v_ref, vb_ref, z_ref):
        x = x_ref[...]
        u = jnp.tanh(
            jnp.dot(x, w_ref[...], preferred_element_type=jnp.float32)
            + b_ref[...])
        z = jnp.sum(u * v_ref[...], axis=-1) + vb_ref[0, 0]
        z_ref[...] = z.reshape(ZROWS, 128)

    rows = ZROWS * 128
    grid = N // rows
    return pl.pallas_call(
        body,
        grid=(grid,),
        in_specs=[
            pl.BlockSpec((rows, 128), lambda i: (i, 0)),
            pl.BlockSpec((128, A), lambda i: (0, 0)),
            pl.BlockSpec((1, A), lambda i: (0, 0)),
            pl.BlockSpec((1, A), lambda i: (0, 0)),
            pl.BlockSpec((1, 1), lambda i: (0, 0)),
        ],
        out_specs=pl.BlockSpec((ZROWS, 128), lambda i: (i, 0)),
        out_shape=jax.ShapeDtypeStruct((N // 128, 128), jnp.float32),
    )(x_flat, attW, attb2, attV2, attvb2)


def _attn_softmax_mask(attn, ids):
    def body(attn_ref, ids_ref, alpha_ref, msk_ref):
        a = attn_ref[...]
        mx = jnp.max(a, axis=-1, keepdims=True)
        ex = jnp.exp(a - mx)
        alpha_ref[...] = ex / jnp.sum(ex, axis=-1, keepdims=True)
        msk_ref[...] = (ids_ref[...] != 0).astype(jnp.float32)

    return pl.pallas_call(
        body,
        out_shape=(jax.ShapeDtypeStruct((B, S), jnp.float32),
                   jax.ShapeDtypeStruct((B, S), jnp.float32)),
    )(attn, ids)


TCH = 8
NT = S // TCH


def _bilstm_head(x_tm, alpha4, msk4, W_f, U_f, bf2, W_b, U_b, bb2,
                 dWfc, dWbc, db2):
    def body(xf_ref, xb_ref, af_ref, ab_ref, mf_ref, mb_ref,
             wf_ref, uf_ref, bf_ref, wb_ref, ub_ref, bb_ref,
             dwf_ref, dwb_ref, db_ref, out_ref,
             hf_s, cf_s, of_s, hb_s, cb_s, ob_s, acc_s):
        c = pl.program_id(0)

        @pl.when(c == 0)
        def _init():
            zz = jnp.zeros((B, H), jnp.float32)
            hf_s[...] = zz
            cf_s[...] = zz
            of_s[...] = zz
            hb_s[...] = zz
            cb_s[...] = zz
            ob_s[...] = zz
            acc_s[...] = jnp.zeros((B, C), jnp.float32)

        uf = uf_ref[...]
        ub = ub_ref[...]
        af = af_ref[0]
        ab = ab_ref[0]
        mf = mf_ref[0]
        mb = mb_ref[0]

        bf16 = jnp.bfloat16
        xf = xf_ref[...].reshape(TCH * B, 128)[:, :E].astype(bf16)
        xwf = (jnp.dot(xf, wf_ref[...], preferred_element_type=jnp.float32)
               + bf_ref[...]).reshape(TCH, B, 4 * H)
        xb = xb_ref[...].reshape(TCH * B, 128)[:, :E].astype(bf16)
        xwb = (jnp.dot(xb, wb_ref[...], preferred_element_type=jnp.float32)
               + bb_ref[...]).reshape(TCH, B, 4 * H)

        hf, cf, of = hf_s[...], cf_s[...], of_s[...]
        hb, cb, ob = hb_s[...], cb_s[...], ob_s[...]

        def gates(zg, cc):
            ig = jax.nn.sigmoid(zg[:, 0:H])
            fg = jax.nn.sigmoid(zg[:, H:2 * H])
            gg = jnp.tanh(zg[:, 2 * H:3 * H])
            og = jax.nn.sigmoid(zg[:, 3 * H:4 * H])
            c_new = fg * cc + ig * gg
            h_new = og * jnp.tanh(c_new)
            return h_new, c_new

        parts_f = []
        parts_b = [None] * TCH
        for tl in range(TCH):
            zgf = xwf[tl] + jnp.dot(hf.astype(bf16), uf,
                                    preferred_element_type=jnp.float32)
            hfn, cfn = gates(zgf, cf)
            m = mf[:, tl:tl + 1] > 0.0
            hf = jnp.where(m, hfn, hf)
            cf = jnp.where(m, cfn, cf)
            of = jnp.where(m, hfn, of)
            parts_f.append(of * af[:, tl:tl + 1])
            tr = TCH - 1 - tl
            zgb = xwb[tr] + jnp.dot(hb.astype(bf16), ub,
                                    preferred_element_type=jnp.float32)
            hbn, cbn = gates(zgb, cb)
            mr = mb[:, tr:tr + 1] > 0.0
            hb = jnp.where(mr, hbn, hb)
            cb = jnp.where(mr, cbn, cb)
            ob = jnp.where(mr, hbn, ob)
            parts_b[tr] = ob * ab[:, tr:tr + 1]

        ofcat = jnp.concatenate(parts_f, axis=1).astype(bf16)
        obcat = jnp.concatenate(parts_b, axis=1).astype(bf16)
        acc = (acc_s[...]
               + jnp.dot(ofcat, dwf_ref[0], preferred_element_type=jnp.float32)
               + jnp.dot(obcat, dwb_ref[0], preferred_element_type=jnp.float32))

        hf_s[...], cf_s[...], of_s[...] = hf, cf, of
        hb_s[...], cb_s[...], ob_s[...] = hb, cb, ob
        acc_s[...] = acc

        @pl.when(c == NT - 1)
        def _fin():
            logits = acc + db_ref[...]
            lm = jnp.max(logits, axis=-1, keepdims=True)
            el = jnp.exp(logits - lm)
            out_ref[...] = el / jnp.sum(el, axis=-1, keepdims=True)

    return pl.pallas_call(
        body,
        grid=(NT,),
        in_specs=[
            pl.BlockSpec((TCH, B, 128), lambda c: (c, 0, 0)),
            pl.BlockSpec((TCH, B, 128), lambda c: (NT - 1 - c, 0, 0)),
            pl.BlockSpec((1, B, TCH), lambda c: (c, 0, 0)),
            pl.BlockSpec((1, B, TCH), lambda c: (NT - 1 - c, 0, 0)),
            pl.BlockSpec((1, B, TCH), lambda c: (c, 0, 0)),
            pl.BlockSpec((1, B, TCH), lambda c: (NT - 1 - c, 0, 0)),
            pl.BlockSpec((E, 4 * H), lambda c: (0, 0)),
            pl.BlockSpec((H, 4 * H), lambda c: (0, 0)),
            pl.BlockSpec((1, 4 * H), lambda c: (0, 0)),
            pl.BlockSpec((E, 4 * H), lambda c: (0, 0)),
            pl.BlockSpec((H, 4 * H), lambda c: (0, 0)),
            pl.BlockSpec((1, 4 * H), lambda c: (0, 0)),
            pl.BlockSpec((1, TCH * H, C), lambda c: (c, 0, 0)),
            pl.BlockSpec((1, TCH * H, C), lambda c: (NT - 1 - c, 0, 0)),
            pl.BlockSpec((1, C), lambda c: (0, 0)),
        ],
        out_specs=pl.BlockSpec((B, C), lambda c: (0, 0)),
        out_shape=jax.ShapeDtypeStruct((B, C), jnp.float32),
        scratch_shapes=[pltpu.VMEM((B, H), jnp.float32)] * 6
        + [pltpu.VMEM((B, C), jnp.float32)],
    )(x_tm, x_tm, alpha4, alpha4, msk4, msk4,
      W_f, U_f, bf2, W_b, U_b, bb2, dWfc, dWbc, db2)


def kernel(inputs, emb, W_f, U_f, b_f, W_b, U_b, b_b,
           attW, attb, attV, attvb, dW, db):
    ids_tm = inputs.T
    idx3d = ids_tm.reshape(NW, NCHUNK, CHUNK)
    embp = jnp.pad(emb, ((0, 0), (0, 128 - E)))
    x2 = _sc_gather(idx3d, embp)
    x_flat = x2.reshape(N, 128)
    attWp = jnp.pad(attW, ((0, 128 - E), (0, 0)))
    z2d = _attn_scores(x_flat, attWp, attb.reshape(1, A),
                       attV.reshape(1, A), attvb.reshape(1, 1))
    z_tm = z2d.reshape(S, B)
    attn = z_tm.T.reshape(-1).reshape(S, B).T
    alpha, mskf = _attn_softmax_mask(attn, inputs)
    alpha4 = alpha.reshape(B, NT, TCH).transpose(1, 0, 2)
    msk4 = mskf.reshape(B, NT, TCH).transpose(1, 0, 2)
    x_tm = x2.reshape(S, B, 128)
    dW3 = dW.reshape(S, 2 * H, C)
    dWfc = dW3[:, :H, :].reshape(NT, TCH * H, C)
    dWbc = dW3[:, H:, :].reshape(NT, TCH * H, C)
    bf16 = jnp.bfloat16
    out = _bilstm_head(x_tm, alpha4, msk4,
                       W_f.astype(bf16), U_f.astype(bf16), b_f.reshape(1, -1),
                       W_b.astype(bf16), U_b.astype(bf16), b_b.reshape(1, -1),
                       dWfc.astype(bf16), dWbc.astype(bf16), db.reshape(1, -1))
    return out


# speedup vs baseline: 1.8613x; 1.0186x over previous
import functools

import jax
import jax.numpy as jnp
from jax import lax
from jax.experimental import pallas as pl
from jax.experimental.pallas import tpu as pltpu
from jax.experimental.pallas import tpu_sc as plsc

B, S, E, H, A, C = 1024, 200, 64, 64, 64, 16
N = B * S
NW = 32
ROWS_PER_W = N // NW
CHUNK = 128
NCHUNK = ROWS_PER_W // CHUNK
BB = 128
ZROWS = 64


def _sc_gather(idx3d, table):
    mesh = plsc.VectorSubcoreMesh(core_axis_name="c", subcore_axis_name="s")

    @functools.partial(
        pl.kernel,
        mesh=mesh,
        out_type=jax.ShapeDtypeStruct((NW * NCHUNK, CHUNK, 128), jnp.float32),
        scratch_types=[
            pltpu.VMEM((NCHUNK, CHUNK), jnp.int32),
            pltpu.VMEM((CHUNK, 128), jnp.float32),
            pltpu.VMEM((CHUNK, 128), jnp.float32),
            pltpu.SemaphoreType.DMA,
            pltpu.SemaphoreType.DMA,
            pltpu.SemaphoreType.DMA,
            pltpu.SemaphoreType.DMA,
        ],
    )
    def k(idx_hbm, table_hbm, out_hbm, idx_v, rows_a, rows_b, gs_a, gs_b,
          os_a, os_b):
        wid = lax.axis_index("s") * 2 + lax.axis_index("c")
        base = wid * NCHUNK
        pltpu.sync_copy(idx_hbm.at[wid], idx_v)

        pltpu.async_copy(table_hbm.at[idx_v.at[0]], rows_a, gs_a)

        def body(i, carry):
            j = i * 2

            def stage(jj, rows, gsem, osem, rows_nxt, gsem_nxt, osem_nxt):
                pltpu.make_async_copy(
                    table_hbm.at[idx_v.at[jj]], rows, gsem).wait()

                @pl.when(jj + 1 < NCHUNK)
                def _():
                    @pl.when(jj >= 1)
                    def _():
                        pltpu.make_async_copy(
                            rows_nxt, out_hbm.at[base], osem_nxt).wait()
                    pltpu.async_copy(
                        table_hbm.at[idx_v.at[jj + 1]], rows_nxt, gsem_nxt)

                pltpu.async_copy(rows, out_hbm.at[base + jj], osem)

            stage(j, rows_a, gs_a, os_a, rows_b, gs_b, os_b)
            stage(j + 1, rows_b, gs_b, os_b, rows_a, gs_a, os_a)
            return carry

        lax.fori_loop(0, NCHUNK // 2, body, 0)
        pltpu.make_async_copy(rows_a, out_hbm.at[base], os_a).wait()
        pltpu.make_async_copy(rows_b, out_hbm.at[base], os_b).wait()

    return k(idx3d, table)


def _attn_scores(x_flat, attW, attb2, attV2, attvb2):
    def body(x_ref, w_ref, b_ref, v_ref, vb_ref, z_ref):
        x = x_ref[...]
        u = jnp.tanh(
            jnp.dot(x, w_ref[...], preferred_element_type=jnp.float32)
            + b_ref[...])
        z = jnp.sum(u * v_ref[...], axis=-1) + vb_ref[0, 0]
        z_ref[...] = z.reshape(ZROWS, 128)

    rows = ZROWS * 128
    grid = N // rows
    return pl.pallas_call(
        body,
        grid=(grid,),
        in_specs=[
            pl.BlockSpec((rows, 128), lambda i: (i, 0)),
            pl.BlockSpec((128, A), lambda i: (0, 0)),
            pl.BlockSpec((1, A), lambda i: (0, 0)),
            pl.BlockSpec((1, A), lambda i: (0, 0)),
            pl.BlockSpec((1, 1), lambda i: (0, 0)),
        ],
        out_specs=pl.BlockSpec((ZROWS, 128), lambda i: (i, 0)),
        out_shape=jax.ShapeDtypeStruct((N // 128, 128), jnp.float32),
    )(x_flat, attW, attb2, attV2, attvb2)


def _attn_softmax_mask(attn, ids):
    def body(attn_ref, ids_ref, alpha_ref, msk_ref):
        a = attn_ref[...]
        mx = jnp.max(a, axis=-1, keepdims=True)
        ex = jnp.exp(a - mx)
        alpha_ref[...] = ex / jnp.sum(ex, axis=-1, keepdims=True)
        msk_ref[...] = (ids_ref[...] != 0).astype(jnp.float32)

    return pl.pallas_call(
        body,
        out_shape=(jax.ShapeDtypeStruct((B, S), jnp.float32),
                   jax.ShapeDtypeStruct((B, S), jnp.float32)),
    )(attn, ids)


TCH = 4
NT = S // TCH


def _bilstm_head(x_tm, alpha4, msk4, W_f, U_f, bf2, W_b, U_b, bb2,
                 dWfc, dWbc, db2):
    def body(xf_ref, xb_ref, af_ref, ab_ref, mf_ref, mb_ref,
             wf_ref, uf_ref, bf_ref, wb_ref, ub_ref, bb_ref,
             dwf_ref, dwb_ref, db_ref, out_ref,
             hf_s, cf_s, of_s, hb_s, cb_s, ob_s, acc_s):
        c = pl.program_id(0)

        @pl.when(c == 0)
        def _init():
            zz = jnp.zeros((B, H), jnp.float32)
            hf_s[...] = zz
            cf_s[...] = zz
            of_s[...] = zz
            hb_s[...] = zz
            cb_s[...] = zz
            ob_s[...] = zz
            acc_s[...] = jnp.zeros((B, C), jnp.float32)

        uf = uf_ref[...]
        ub = ub_ref[...]
        af = af_ref[0]
        ab = ab_ref[0]
        mf = mf_ref[0]
        mb = mb_ref[0]

        bf16 = jnp.bfloat16
        xf = xf_ref[...].reshape(TCH * B, 128)[:, :E].astype(bf16)
        xwf = (jnp.dot(xf, wf_ref[...], preferred_element_type=jnp.float32)
               + bf_ref[...]).reshape(TCH, B, 512)
        xb = xb_ref[...].reshape(TCH * B, 128)[:, :E].astype(bf16)
        xwb = (jnp.dot(xb, wb_ref[...], preferred_element_type=jnp.float32)
               + bb_ref[...]).reshape(TCH, B, 512)

        hf, cf, of = hf_s[...], cf_s[...], of_s[...]
        hb, cb, ob = hb_s[...], cb_s[...], ob_s[...]

        def gates(zg, cc):
            ig = jax.nn.sigmoid(zg[:, 0:H])
            fg = jax.nn.sigmoid(zg[:, 128:128 + H])
            gg = jnp.tanh(zg[:, 256:256 + H])
            og = jax.nn.sigmoid(zg[:, 384:384 + H])
            c_new = fg * cc + ig * gg
            h_new = og * jnp.tanh(c_new)
            return h_new, c_new

        parts_f = []
        parts_b = [None] * TCH
        for tl in range(TCH):
            zgf = xwf[tl] + jnp.dot(hf.astype(bf16), uf,
                                    preferred_element_type=jnp.float32)
            hfn, cfn = gates(zgf, cf)
            m = mf[:, tl:tl + 1] > 0.0
            hf = jnp.where(m, hfn, hf)
            cf = jnp.where(m, cfn, cf)
            of = jnp.where(m, hfn, of)
            parts_f.append(of * af[:, tl:tl + 1])
            tr = TCH - 1 - tl
            zgb = xwb[tr] + jnp.dot(hb.astype(bf16), ub,
                                    preferred_element_type=jnp.float32)
            hbn, cbn = gates(zgb, cb)
            mr = mb[:, tr:tr + 1] > 0.0
            hb = jnp.where(mr, hbn, hb)
            cb = jnp.where(mr, cbn, cb)
            ob = jnp.where(mr, hbn, ob)
            parts_b[tr] = ob * ab[:, tr:tr + 1]

        ofcat = jnp.concatenate(parts_f, axis=1).astype(bf16)
        obcat = jnp.concatenate(parts_b, axis=1).astype(bf16)
        acc = (acc_s[...]
               + jnp.dot(ofcat, dwf_ref[0], preferred_element_type=jnp.float32)
               + jnp.dot(obcat, dwb_ref[0], preferred_element_type=jnp.float32))

        hf_s[...], cf_s[...], of_s[...] = hf, cf, of
        hb_s[...], cb_s[...], ob_s[...] = hb, cb, ob
        acc_s[...] = acc

        @pl.when(c == NT - 1)
        def _fin():
            logits = acc + db_ref[...]
            lm = jnp.max(logits, axis=-1, keepdims=True)
            el = jnp.exp(logits - lm)
            out_ref[...] = el / jnp.sum(el, axis=-1, keepdims=True)

    return pl.pallas_call(
        body,
        grid=(NT,),
        in_specs=[
            pl.BlockSpec((TCH, B, 128), lambda c: (c, 0, 0)),
            pl.BlockSpec((TCH, B, 128), lambda c: (NT - 1 - c, 0, 0)),
            pl.BlockSpec((1, B, TCH), lambda c: (c, 0, 0)),
            pl.BlockSpec((1, B, TCH), lambda c: (NT - 1 - c, 0, 0)),
            pl.BlockSpec((1, B, TCH), lambda c: (c, 0, 0)),
            pl.BlockSpec((1, B, TCH), lambda c: (NT - 1 - c, 0, 0)),
            pl.BlockSpec((E, 512), lambda c: (0, 0)),
            pl.BlockSpec((H, 512), lambda c: (0, 0)),
            pl.BlockSpec((1, 512), lambda c: (0, 0)),
            pl.BlockSpec((E, 512), lambda c: (0, 0)),
            pl.BlockSpec((H, 512), lambda c: (0, 0)),
            pl.BlockSpec((1, 512), lambda c: (0, 0)),
            pl.BlockSpec((1, TCH * H, C), lambda c: (c, 0, 0)),
            pl.BlockSpec((1, TCH * H, C), lambda c: (NT - 1 - c, 0, 0)),
            pl.BlockSpec((1, C), lambda c: (0, 0)),
        ],
        out_specs=pl.BlockSpec((B, C), lambda c: (0, 0)),
        out_shape=jax.ShapeDtypeStruct((B, C), jnp.float32),
        scratch_shapes=[pltpu.VMEM((B, H), jnp.float32)] * 6
        + [pltpu.VMEM((B, C), jnp.float32)],
    )(x_tm, x_tm, alpha4, alpha4, msk4, msk4,
      W_f, U_f, bf2, W_b, U_b, bb2, dWfc, dWbc, db2)


def kernel(inputs, emb, W_f, U_f, b_f, W_b, U_b, b_b,
           attW, attb, attV, attvb, dW, db):
    ids_tm = inputs.T
    idx3d = ids_tm.reshape(NW, NCHUNK, CHUNK)
    embp = jnp.pad(emb, ((0, 0), (0, 128 - E)))
    x2 = _sc_gather(idx3d, embp)
    x_flat = x2.reshape(N, 128)
    attWp = jnp.pad(attW, ((0, 128 - E), (0, 0)))
    z2d = _attn_scores(x_flat, attWp, attb.reshape(1, A),
                       attV.reshape(1, A), attvb.reshape(1, 1))
    z_tm = z2d.reshape(S, B)
    attn = z_tm.T.reshape(-1).reshape(S, B).T
    alpha, mskf = _attn_softmax_mask(attn, inputs)
    alpha4 = alpha.reshape(B, NT, TCH).transpose(1, 0, 2)
    msk4 = mskf.reshape(B, NT, TCH).transpose(1, 0, 2)
    x_tm = x2.reshape(S, B, 128)
    dW3 = dW.reshape(S, 2 * H, C)
    dWfc = dW3[:, :H, :].reshape(NT, TCH * H, C)
    dWbc = dW3[:, H:, :].reshape(NT, TCH * H, C)
    bf16 = jnp.bfloat16

    def gate_pad(w):
        w4 = w.reshape(-1, 4, H)
        return jnp.pad(w4, ((0, 0), (0, 0), (0, 128 - H))).reshape(-1, 512)

    out = _bilstm_head(x_tm, alpha4, msk4,
                       gate_pad(W_f).astype(bf16), gate_pad(U_f).astype(bf16),
                       gate_pad(b_f.reshape(1, -1)),
                       gate_pad(W_b).astype(bf16), gate_pad(U_b).astype(bf16),
                       gate_pad(b_b.reshape(1, -1)),
                       dWfc.astype(bf16), dWbc.astype(bf16), db.reshape(1, -1))
    return out


# speedup vs baseline: 1.9075x; 1.0248x over previous
import functools

import jax
import jax.numpy as jnp
from jax import lax
from jax.experimental import pallas as pl
from jax.experimental.pallas import tpu as pltpu
from jax.experimental.pallas import tpu_sc as plsc

B, S, E, H, A, C = 1024, 200, 64, 64, 64, 16
N = B * S
NW = 32
ROWS_PER_W = N // NW
CHUNK = 128
NCHUNK = ROWS_PER_W // CHUNK
BB = 128
ZROWS = 64


def _sc_gather(idx3d, table):
    mesh = plsc.VectorSubcoreMesh(core_axis_name="c", subcore_axis_name="s")

    @functools.partial(
        pl.kernel,
        mesh=mesh,
        out_type=jax.ShapeDtypeStruct((NW * NCHUNK, CHUNK, 128), jnp.float32),
        scratch_types=[
            pltpu.VMEM((NCHUNK, CHUNK), jnp.int32),
            pltpu.VMEM((CHUNK, 128), jnp.float32),
            pltpu.VMEM((CHUNK, 128), jnp.float32),
            pltpu.SemaphoreType.DMA,
            pltpu.SemaphoreType.DMA,
            pltpu.SemaphoreType.DMA,
            pltpu.SemaphoreType.DMA,
        ],
    )
    def k(idx_hbm, table_hbm, out_hbm, idx_v, rows_a, rows_b, gs_a, gs_b,
          os_a, os_b):
        wid = lax.axis_index("s") * 2 + lax.axis_index("c")
        base = wid * NCHUNK
        pltpu.sync_copy(idx_hbm.at[wid], idx_v)

        pltpu.async_copy(table_hbm.at[idx_v.at[0]], rows_a, gs_a)

        def body(i, carry):
            j = i * 2

            def stage(jj, rows, gsem, osem, rows_nxt, gsem_nxt, osem_nxt):
                pltpu.make_async_copy(
                    table_hbm.at[idx_v.at[jj]], rows, gsem).wait()

                @pl.when(jj + 1 < NCHUNK)
                def _():
                    @pl.when(jj >= 1)
                    def _():
                        pltpu.make_async_copy(
                            rows_nxt, out_hbm.at[base], osem_nxt).wait()
                    pltpu.async_copy(
                        table_hbm.at[idx_v.at[jj + 1]], rows_nxt, gsem_nxt)

                pltpu.async_copy(rows, out_hbm.at[base + jj], osem)

            stage(j, rows_a, gs_a, os_a, rows_b, gs_b, os_b)
            stage(j + 1, rows_b, gs_b, os_b, rows_a, gs_a, os_a)
            return carry

        lax.fori_loop(0, NCHUNK // 2, body, 0)
        pltpu.make_async_copy(rows_a, out_hbm.at[base], os_a).wait()
        pltpu.make_async_copy(rows_b, out_hbm.at[base], os_b).wait()

    return k(idx3d, table)


def _attn_scores(x_flat, attW, attb2, attV2, attvb2):
    def body(x_ref, w_ref, b_ref, v_ref, vb_ref, z_ref):
        x = x_ref[...]
        u = jnp.tanh(
            jnp.dot(x, w_ref[...], preferred_element_type=jnp.float32)
            + b_ref[...])
        z = jnp.sum(u * v_ref[...], axis=-1) + vb_ref[0, 0]
        z_ref[...] = z.reshape(ZROWS, 128)

    rows = ZROWS * 128
    grid = N // rows
    return pl.pallas_call(
        body,
        grid=(grid,),
        in_specs=[
            pl.BlockSpec((rows, 128), lambda i: (i, 0)),
            pl.BlockSpec((128, A), lambda i: (0, 0)),
            pl.BlockSpec((1, A), lambda i: (0, 0)),
            pl.BlockSpec((1, A), lambda i: (0, 0)),
            pl.BlockSpec((1, 1), lambda i: (0, 0)),
        ],
        out_specs=pl.BlockSpec((ZROWS, 128), lambda i: (i, 0)),
        out_shape=jax.ShapeDtypeStruct((N // 128, 128), jnp.float32),
    )(x_flat, attW, attb2, attV2, attvb2)


def _attn_softmax_mask(attn, ids):
    def body(attn_ref, ids_ref, alpha_ref, msk_ref):
        a = attn_ref[...]
        mx = jnp.max(a, axis=-1, keepdims=True)
        ex = jnp.exp(a - mx)
        alpha_ref[...] = ex / jnp.sum(ex, axis=-1, keepdims=True)
        msk_ref[...] = (ids_ref[...] != 0).astype(jnp.float32)

    return pl.pallas_call(
        body,
        out_shape=(jax.ShapeDtypeStruct((B, S), jnp.float32),
                   jax.ShapeDtypeStruct((B, S), jnp.float32)),
    )(attn, ids)


TCH = 4
NT = S // TCH


def _bilstm_head(x_tm, alpha4, msk4, W_f, U_f, bf2, W_b, U_b, bb2,
                 dWfc, dWbc, db2):
    def body(xf_ref, xb_ref, af_ref, ab_ref, mf_ref, mb_ref,
             wf_ref, uf_ref, bf_ref, wb_ref, ub_ref, bb_ref,
             dwf_ref, dwb_ref, db_ref, out_ref,
             hf_s, cf_s, of_s, hb_s, cb_s, ob_s, acc_s):
        c = pl.program_id(0)

        @pl.when(c == 0)
        def _init():
            zz = jnp.zeros((B, H), jnp.float32)
            hf_s[...] = zz
            cf_s[...] = zz
            of_s[...] = zz
            hb_s[...] = zz
            cb_s[...] = zz
            ob_s[...] = zz
            acc_s[...] = jnp.zeros((B, C), jnp.float32)

        uf = uf_ref[...]
        ub = ub_ref[...]
        af = af_ref[0]
        ab = ab_ref[0]
        mf = mf_ref[0]
        mb = mb_ref[0]

        bf16 = jnp.bfloat16
        xf = xf_ref[...].reshape(TCH * B, 128)[:, :E].astype(bf16)
        xwf = (jnp.dot(xf, wf_ref[...], preferred_element_type=jnp.float32)
               + bf_ref[...]).astype(bf16).reshape(TCH, B, 512)
        xb = xb_ref[...].reshape(TCH * B, 128)[:, :E].astype(bf16)
        xwb = (jnp.dot(xb, wb_ref[...], preferred_element_type=jnp.float32)
               + bb_ref[...]).astype(bf16).reshape(TCH, B, 512)

        hf, cf, of = hf_s[...], cf_s[...], of_s[...]
        hb, cb, ob = hb_s[...], cb_s[...], ob_s[...]

        def sg(v):
            return 0.5 * jnp.tanh(0.5 * v) + 0.5

        def gates(zg, cc):
            ig = sg(zg[:, 0:H])
            fg = sg(zg[:, 128:128 + H])
            gg = jnp.tanh(zg[:, 256:256 + H])
            og = sg(zg[:, 384:384 + H])
            c_new = fg * cc + ig * gg
            h_new = og * jnp.tanh(c_new)
            return h_new, c_new

        parts_f = []
        parts_b = [None] * TCH
        for tl in range(TCH):
            zgf = (xwf[tl].astype(jnp.float32)
                   + jnp.dot(hf.astype(bf16), uf,
                             preferred_element_type=jnp.float32))
            hfn, cfn = gates(zgf, cf)
            m = mf[:, tl:tl + 1] > 0.0
            hf = jnp.where(m, hfn, hf)
            cf = jnp.where(m, cfn, cf)
            of = jnp.where(m, hfn, of)
            parts_f.append(of * af[:, tl:tl + 1])
            tr = TCH - 1 - tl
            zgb = (xwb[tr].astype(jnp.float32)
                   + jnp.dot(hb.astype(bf16), ub,
                             preferred_element_type=jnp.float32))
            hbn, cbn = gates(zgb, cb)
            mr = mb[:, tr:tr + 1] > 0.0
            hb = jnp.where(mr, hbn, hb)
            cb = jnp.where(mr, cbn, cb)
            ob = jnp.where(mr, hbn, ob)
            parts_b[tr] = ob * ab[:, tr:tr + 1]

        ofcat = jnp.concatenate(parts_f, axis=1).astype(bf16)
        obcat = jnp.concatenate(parts_b, axis=1).astype(bf16)
        acc = (acc_s[...]
               + jnp.dot(ofcat, dwf_ref[0], preferred_element_type=jnp.float32)
               + jnp.dot(obcat, dwb_ref[0], preferred_element_type=jnp.float32))

        hf_s[...], cf_s[...], of_s[...] = hf, cf, of
        hb_s[...], cb_s[...], ob_s[...] = hb, cb, ob
        acc_s[...] = acc

        @pl.when(c == NT - 1)
        def _fin():
            logits = acc + db_ref[...]
            lm = jnp.max(logits, axis=-1, keepdims=True)
            el = jnp.exp(logits - lm)
            out_ref[...] = el / jnp.sum(el, axis=-1, keepdims=True)

    return pl.pallas_call(
        body,
        grid=(NT,),
        in_specs=[
            pl.BlockSpec((TCH, B, 128), lambda c: (c, 0, 0)),
            pl.BlockSpec((TCH, B, 128), lambda c: (NT - 1 - c, 0, 0)),
            pl.BlockSpec((1, B, TCH), lambda c: (c, 0, 0)),
            pl.BlockSpec((1, B, TCH), lambda c: (NT - 1 - c, 0, 0)),
            pl.BlockSpec((1, B, TCH), lambda c: (c, 0, 0)),
            pl.BlockSpec((1, B, TCH), lambda c: (NT - 1 - c, 0, 0)),
            pl.BlockSpec((E, 512), lambda c: (0, 0)),
            pl.BlockSpec((H, 512), lambda c: (0, 0)),
            pl.BlockSpec((1, 512), lambda c: (0, 0)),
            pl.BlockSpec((E, 512), lambda c: (0, 0)),
            pl.BlockSpec((H, 512), lambda c: (0, 0)),
            pl.BlockSpec((1, 512), lambda c: (0, 0)),
            pl.BlockSpec((1, TCH * H, C), lambda c: (c, 0, 0)),
            pl.BlockSpec((1, TCH * H, C), lambda c: (NT - 1 - c, 0, 0)),
            pl.BlockSpec((1, C), lambda c: (0, 0)),
        ],
        out_specs=pl.BlockSpec((B, C), lambda c: (0, 0)),
        out_shape=jax.ShapeDtypeStruct((B, C), jnp.float32),
        scratch_shapes=[pltpu.VMEM((B, H), jnp.float32)] * 6
        + [pltpu.VMEM((B, C), jnp.float32)],
    )(x_tm, x_tm, alpha4, alpha4, msk4, msk4,
      W_f, U_f, bf2, W_b, U_b, bb2, dWfc, dWbc, db2)


def kernel(inputs, emb, W_f, U_f, b_f, W_b, U_b, b_b,
           attW, attb, attV, attvb, dW, db):
    ids_tm = inputs.T
    idx3d = ids_tm.reshape(NW, NCHUNK, CHUNK)
    embp = jnp.pad(emb, ((0, 0), (0, 128 - E)))
    x2 = _sc_gather(idx3d, embp)
    x_flat = x2.reshape(N, 128)
    attWp = jnp.pad(attW, ((0, 128 - E), (0, 0)))
    z2d = _attn_scores(x_flat, attWp, attb.reshape(1, A),
                       attV.reshape(1, A), attvb.reshape(1, 1))
    z_tm = z2d.reshape(S, B)
    attn = z_tm.T.reshape(-1).reshape(S, B).T
    alpha, mskf = _attn_softmax_mask(attn, inputs)
    alpha4 = alpha.reshape(B, NT, TCH).transpose(1, 0, 2)
    msk4 = mskf.reshape(B, NT, TCH).transpose(1, 0, 2)
    x_tm = x2.reshape(S, B, 128)
    dW3 = dW.reshape(S, 2 * H, C)
    dWfc = dW3[:, :H, :].reshape(NT, TCH * H, C)
    dWbc = dW3[:, H:, :].reshape(NT, TCH * H, C)
    bf16 = jnp.bfloat16

    def gate_pad(w):
        w4 = w.reshape(-1, 4, H)
        return jnp.pad(w4, ((0, 0), (0, 0), (0, 128 - H))).reshape(-1, 512)

    out = _bilstm_head(x_tm, alpha4, msk4,
                       gate_pad(W_f).astype(bf16), gate_pad(U_f).astype(bf16),
                       gate_pad(b_f.reshape(1, -1)),
                       gate_pad(W_b).astype(bf16), gate_pad(U_b).astype(bf16),
                       gate_pad(b_b.reshape(1, -1)),
                       dWfc.astype(bf16), dWbc.astype(bf16), db.reshape(1, -1))
    return out
